# baseline (speedup 1.0000x reference)
import jax
import jax.numpy as jnp
from jax.experimental import pallas as pl
from jax.experimental.pallas import tpu as pltpu

_MAXREL = 14
_L = 1025
_R = 32


def _body(tv_ref, th_ref, out_ref, hh_ref):
    g = pl.program_id(0)
    sub = jax.lax.broadcasted_iota(jnp.int32, (32, _L), 0)
    col = jax.lax.broadcasted_iota(jnp.int32, (32, _L), 1)

    @pl.when(g == 0)
    def _init_h_pattern():
        jm = (col - 1) & 31
        for r in range(_R):
            hidx = jnp.where(
                col == 0, 0,
                jnp.clip(jm - ((r + 31) & 31), -_MAXREL, _MAXREL) + _MAXREL + 1)
            ohh = (hidx == sub).astype(jnp.float32)
            hh_ref[r] = jnp.dot(th_ref[...], ohh,
                                preferred_element_type=jnp.float32)

    kb = (col - 1) >> 5

    def vrow(t):
        vidx = jnp.where(
            col == 0, 0,
            jnp.clip(kb - t, -_MAXREL, _MAXREL) + _MAXREL + 1)
        ohv = (vidx == sub).astype(jnp.float32)
        return jnp.dot(tv_ref[...], ohv, preferred_element_type=jnp.float32)

    out_ref[...] = hh_ref[...] + vrow(g)[None]

    @pl.when(g == 0)
    def _row0_edge():
        u = tv_ref[:, 0:1] + th_ref[:, 0:1]
        out_ref[0] = jnp.broadcast_to(u, (64, _L))

    @pl.when(g > 0)
    def _row0_prev():
        out_ref[0] = hh_ref[0] + vrow(g - 1)


def kernel(emb_table_v, emb_table_h, length_q, length_k):
    del length_q, length_k
    tv = jnp.zeros((64, 32), jnp.float32).at[:, :30].set(emb_table_v.T)
    th = jnp.zeros((64, 32), jnp.float32).at[:, :30].set(emb_table_h.T)
    out_t = pl.pallas_call(
        _body,
        grid=(33,),
        in_specs=[
            pl.BlockSpec((64, 32), lambda g: (0, 0)),
            pl.BlockSpec((64, 32), lambda g: (0, 0)),
        ],
        out_specs=pl.BlockSpec((_R, 64, _L), lambda g: (g, 0, 0)),
        out_shape=jax.ShapeDtypeStruct((_L, 64, _L), jnp.float32),
        scratch_shapes=[pltpu.VMEM((_R, 64, _L), jnp.float32)],
    )(tv, th)
    return jnp.transpose(out_t, (0, 2, 1))
